# Initial kernel scaffold; baseline (speedup 1.0000x reference)
import functools

import jax
import jax.numpy as jnp
from jax import lax
from jax.experimental import pallas as pl
from jax.experimental.pallas import tpu as pltpu
from jax.experimental.pallas import tpu_sc as plsc

L = 16


def kernel(x, idx):
    B, Ch, C = x.shape
    K, G = idx.shape
    N = B * Ch
    X = x.reshape(N, C)
    idxT = idx.T

    info = plsc.get_sparse_core_info()
    NC, NS = info.num_cores, info.num_subcores
    NW = NC * NS
    rows_per_w = N // NW
    R = 128
    nchunk = rows_per_w // R
    ngroups = K // L
    scale = 1.0 / G

    mesh = plsc.VectorSubcoreMesh(core_axis_name="c", subcore_axis_name="s")

    @functools.partial(
        pl.kernel,
        mesh=mesh,
        out_type=jax.ShapeDtypeStruct((N, K), jnp.float32),
        scratch_types=[
            pltpu.VMEM((R, C), jnp.float32),
            pltpu.VMEM((R, K), jnp.float32),
            pltpu.VMEM((G, K), jnp.int32),
        ],
    )
    def _unpool(x_hbm, idx_hbm, out_hbm, x_v, o_v, idx_v):
        wid = lax.axis_index("s") * NC + lax.axis_index("c")
        base = wid * rows_per_w
        pltpu.sync_copy(idx_hbm, idx_v)
        cols = [[idx_v[j, pl.ds(g * L, L)] for g in range(ngroups)]
                for j in range(G)]

        def chunk_body(cidx, carry):
            row0 = base + cidx * R
            pltpu.sync_copy(x_hbm.at[pl.ds(row0, R)], x_v)

            def row_body(r, carry2):
                rv = jnp.full((L,), r, dtype=jnp.int32)
                for g in range(ngroups):
                    acc = plsc.load_gather(x_v, [rv, cols[0][g]])
                    for j in range(1, G):
                        acc = acc + plsc.load_gather(x_v, [rv, cols[j][g]])
                    o_v[r, pl.ds(g * L, L)] = acc * scale
                return carry2

            lax.fori_loop(0, R, row_body, 0)
            pltpu.sync_copy(o_v, out_hbm.at[pl.ds(row0, R)])
            return carry

        lax.fori_loop(0, nchunk, chunk_body, 0)

    out = _unpool(X, idxT)
    return out.reshape(B, Ch, K)


# speedup vs baseline: 1.7557x; 1.7557x over previous
import functools

import jax
import jax.numpy as jnp
from jax import lax
from jax.experimental import pallas as pl
from jax.experimental.pallas import tpu as pltpu
from jax.experimental.pallas import tpu_sc as plsc

L = 16


def kernel(x, idx):
    B, Ch, C = x.shape
    K, G = idx.shape
    N = B * Ch
    Xf = x.reshape(N * C)
    idxf = idx.T.reshape(G * K)

    info = plsc.get_sparse_core_info()
    NC, NS = info.num_cores, info.num_subcores
    NW = NC * NS
    rows_per_w = N // NW
    R = 128
    nchunk = rows_per_w // R
    ngroups = K // L
    scale = 1.0 / G

    mesh = plsc.VectorSubcoreMesh(core_axis_name="c", subcore_axis_name="s")

    @functools.partial(
        pl.kernel,
        mesh=mesh,
        compiler_params=pltpu.CompilerParams(needs_layout_passes=False),
        out_type=jax.ShapeDtypeStruct((N * K,), jnp.float32),
        scratch_types=[
            pltpu.VMEM((R * C,), jnp.float32),
            pltpu.VMEM((R * C,), jnp.float32),
            pltpu.VMEM((R * K,), jnp.float32),
            pltpu.VMEM((R * K,), jnp.float32),
            pltpu.VMEM((G * K,), jnp.int32),
            pltpu.SemaphoreType.DMA,
            pltpu.SemaphoreType.DMA,
            pltpu.SemaphoreType.DMA,
            pltpu.SemaphoreType.DMA,
        ],
    )
    def _unpool(x_hbm, idx_hbm, out_hbm, x_v0, x_v1, o_v0, o_v1, idx_v,
                si0, si1, so0, so1):
        xb, ob, si, so = [x_v0, x_v1], [o_v0, o_v1], [si0, si1], [so0, so1]
        wid = lax.axis_index("s") * NC + lax.axis_index("c")
        base = wid * rows_per_w
        pltpu.sync_copy(idx_hbm, idx_v)
        cols = [[idx_v[pl.ds(j * K + g * L, L)] for g in range(ngroups)]
                for j in range(G)]

        def in_copy(c, b):
            return pltpu.make_async_copy(
                x_hbm.at[pl.ds((base + c * R) * C, R * C)], xb[b], si[b])

        def out_copy(c, b):
            return pltpu.make_async_copy(
                ob[b], out_hbm.at[pl.ds((base + c * R) * K, R * K)], so[b])

        def compute(x_v, o_v):
            def row_body(r, carry):
                rbase = jnp.full((L,), r * C, dtype=jnp.int32)
                gath = [[plsc.load_gather(x_v, [rbase + cols[j][g]])
                         for g in range(ngroups)] for j in range(G)]
                for g in range(ngroups):
                    acc = gath[0][g]
                    for j in range(1, G):
                        acc = acc + gath[j][g]
                    o_v[pl.ds(r * K + g * L, L)] = acc * scale
                return carry

            lax.fori_loop(0, R, row_body, 0)

        in_copy(0, 0).start()
        in_copy(1, 1).start()

        def super_body(sc, carry):
            for b in range(2):
                c = sc * 2 + b
                in_copy(c, b).wait()

                @pl.when(sc > 0)
                def _():
                    out_copy(c - 2, b).wait()

                compute(xb[b], ob[b])
                out_copy(c, b).start()

                @pl.when(c + 2 < nchunk)
                def _():
                    in_copy(c + 2, b).start()

            return carry

        lax.fori_loop(0, nchunk // 2, super_body, 0)
        out_copy(nchunk - 2, 0).wait()
        out_copy(nchunk - 1, 1).wait()

    out = _unpool(Xf, idxf)
    return out.reshape(B, Ch, K)


# speedup vs baseline: 3.1384x; 1.7876x over previous
import functools

import jax
import jax.numpy as jnp
from jax import lax
from jax.experimental import pallas as pl
from jax.experimental.pallas import tpu as pltpu
from jax.experimental.pallas import tpu_sc as plsc

L = 16


def kernel(x, idx):
    B, Ch, C = x.shape
    K, G = idx.shape
    N = B * Ch
    X2 = x.reshape(N, C)

    info = plsc.get_sparse_core_info()
    NC, NS = info.num_cores, info.num_subcores
    NW = NC * NS
    rows_per_w = N // NW
    R = 128
    nchunk = rows_per_w // R
    ngroups = K // L
    scale = 1.0 / G

    mesh = plsc.VectorSubcoreMesh(core_axis_name="c", subcore_axis_name="s")

    @functools.partial(
        pl.kernel,
        mesh=mesh,
        compiler_params=pltpu.CompilerParams(needs_layout_passes=False),
        out_type=jax.ShapeDtypeStruct((N, K), jnp.float32),
        scratch_types=[
            pltpu.VMEM((R, C), jnp.float32),
            pltpu.VMEM((R, C), jnp.float32),
            pltpu.VMEM((R, K), jnp.float32),
            pltpu.VMEM((R, K), jnp.float32),
            pltpu.VMEM((K, G), jnp.int32),
            pltpu.SemaphoreType.DMA,
            pltpu.SemaphoreType.DMA,
            pltpu.SemaphoreType.DMA,
            pltpu.SemaphoreType.DMA,
        ],
    )
    def _unpool(x_hbm, idx_hbm, out_hbm, x_v0, x_v1, o_v0, o_v1, idx_v,
                si0, si1, so0, so1):
        xb, ob, si, so = [x_v0, x_v1], [o_v0, o_v1], [si0, si1], [so0, so1]
        wid = lax.axis_index("s") * NC + lax.axis_index("c")
        base = wid * rows_per_w
        pltpu.sync_copy(idx_hbm, idx_v)
        lanes = lax.iota(jnp.int32, L)
        cols = [[plsc.load_gather(idx_v,
                                  [lanes + g * L,
                                   jnp.full((L,), j, dtype=jnp.int32)])
                 for g in range(ngroups)] for j in range(G)]

        def in_copy(c, b):
            return pltpu.make_async_copy(
                x_hbm.at[pl.ds(base + c * R, R)], xb[b], si[b])

        def out_copy(c, b):
            return pltpu.make_async_copy(
                ob[b], out_hbm.at[pl.ds(base + c * R, R)], so[b])

        def compute(x_v, o_v):
            def row_body(r, carry):
                rvec = jnp.full((L,), r, dtype=jnp.int32)
                gath = [[plsc.load_gather(x_v, [rvec, cols[j][g]])
                         for g in range(ngroups)] for j in range(G)]
                for g in range(ngroups):
                    acc = gath[0][g]
                    for j in range(1, G):
                        acc = acc + gath[j][g]
                    o_v[r, pl.ds(g * L, L)] = acc * scale
                return carry

            lax.fori_loop(0, R, row_body, 0)

        in_copy(0, 0).start()
        in_copy(1, 1).start()

        def super_body(sc, carry):
            for b in range(2):
                c = sc * 2 + b
                in_copy(c, b).wait()

                @pl.when(sc > 0)
                def _():
                    out_copy(c - 2, b).wait()

                compute(xb[b], ob[b])
                out_copy(c, b).start()

                @pl.when(c + 2 < nchunk)
                def _():
                    in_copy(c + 2, b).start()

            return carry

        lax.fori_loop(0, nchunk // 2, super_body, 0)
        out_copy(nchunk - 2, 0).wait()
        out_copy(nchunk - 1, 1).wait()

    out = _unpool(X2, idx)
    return out.reshape(B, Ch, K)
